# baseline (speedup 1.0000x reference)
import jax
import jax.numpy as jnp
from jax import lax
from jax.experimental import pallas as pl
from jax.experimental.pallas import tpu as pltpu
from jax.experimental.pallas import tpu_sc as plsc

E = 160000
V = 10000
D = 256
DW = D // 2
NC = 2
NS = 16
NW = NC * NS
PER_W = E // NW
C = 192
NFULL = PER_W // C
TAIL = PER_W - NFULL * C
LOOP_CHUNKS = NFULL - 2
G = C // 16
OV_LEN = NFULL * C + 16


def _dot_group(buf, row0, mbase):
    lane = lax.iota(jnp.int32, 16)
    zero = jnp.zeros((16,), jnp.float32)

    def jbody(j, res):
        e = row0 + j
        me = mbase + e
        a0 = zero
        a1 = zero
        for s in range(DW // 32):
            u0 = plsc.bitcast(buf[e, pl.ds(32 * s, 16)], jnp.bfloat16)
            m0 = plsc.bitcast(buf[me, pl.ds(32 * s, 16)], jnp.bfloat16)
            u1 = plsc.bitcast(buf[e, pl.ds(32 * s + 16, 16)], jnp.bfloat16)
            m1 = plsc.bitcast(buf[me, pl.ds(32 * s + 16, 16)], jnp.bfloat16)
            q = u0 * m0 + u1 * m1
            pa, pb = plsc.unpack(q, format=plsc.PackFormat.INTERLEAVED)
            a0 = a0 + pa
            a1 = a1 + pb
        acc = a0 + a1
        for k in (1, 2, 4, 8):
            acc = acc + jnp.take_along_axis(acc, lane ^ k, axis=0,
                                            mode="promise_in_bounds")
        return jnp.where(lane == j, acc, res)

    return lax.fori_loop(0, 16, jbody, zero, unroll=2)


def _compute_chunk(buf, ov, out_off):
    def body(g, carry):
        ov[pl.ds(out_off + g * 16, 16)] = _dot_group(buf, g * 16, C)
        return carry

    lax.fori_loop(0, G, body, 0)


def _sc_body(tab, eli, out, idxall, b0, b1, ov, su0, su1, sm0, sm1):
    c = lax.axis_index("c")
    s = lax.axis_index("s")
    wid = s * NC + c
    base = wid * PER_W

    bufs = (b0, b1)
    sus = (su0, su1)
    sms = (sm0, sm1)

    def fire_rows(k, b):
        off = k * C
        pltpu.async_copy(tab.at[idxall.at[pl.ds(off, C)]],
                         bufs[b].at[pl.ds(0, C)], sus[b])
        pltpu.async_copy(tab.at[idxall.at[pl.ds(PER_W + off, C)]],
                         bufs[b].at[pl.ds(C, C)], sms[b])

    def wait_rows(k, b):
        off = k * C
        pltpu.make_async_copy(tab.at[idxall.at[pl.ds(off, C)]],
                              bufs[b].at[pl.ds(0, C)], sus[b]).wait()
        pltpu.make_async_copy(tab.at[idxall.at[pl.ds(PER_W + off, C)]],
                              bufs[b].at[pl.ds(C, C)], sms[b]).wait()

    pltpu.sync_copy(eli.at[pl.ds(base, PER_W)], idxall.at[pl.ds(0, PER_W)])
    pltpu.sync_copy(eli.at[pl.ds(E + base, PER_W)],
                    idxall.at[pl.ds(PER_W, PER_W)])

    fire_rows(0, 0)

    def pair_body(p, carry):
        for half in range(2):
            k = 2 * p + half
            b = half
            nb = 1 - half
            fire_rows(k + 1, nb)
            wait_rows(k, b)
            _compute_chunk(bufs[b], ov, k * C)
        return carry

    lax.fori_loop(0, LOOP_CHUNKS // 2, pair_body, 0)

    k0 = LOOP_CHUNKS
    k1 = LOOP_CHUNKS + 1
    fire_rows(k1, 1)
    wait_rows(k0, 0)
    _compute_chunk(b0, ov, k0 * C)

    toff = NFULL * C
    pltpu.async_copy(tab.at[idxall.at[pl.ds(toff, TAIL)]],
                     b0.at[pl.ds(0, TAIL)], su0)
    pltpu.async_copy(tab.at[idxall.at[pl.ds(PER_W + toff, TAIL)]],
                     b0.at[pl.ds(TAIL, TAIL)], sm0)

    wait_rows(k1, 1)
    _compute_chunk(b1, ov, k1 * C)

    pltpu.make_async_copy(tab.at[idxall.at[pl.ds(toff, TAIL)]],
                          b0.at[pl.ds(0, TAIL)], su0).wait()
    pltpu.make_async_copy(tab.at[idxall.at[pl.ds(PER_W + toff, TAIL)]],
                          b0.at[pl.ds(TAIL, TAIL)], sm0).wait()
    ov[pl.ds(NFULL * C, 16)] = _dot_group(b0, 0, TAIL)

    pltpu.sync_copy(ov.at[pl.ds(0, PER_W)], out.at[pl.ds(base, PER_W)])


def kernel(x_user, x_movie, edge_label_index):
    mesh = plsc.VectorSubcoreMesh(core_axis_name="c", subcore_axis_name="s")
    run = pl.kernel(
        _sc_body,
        out_type=jax.ShapeDtypeStruct((E,), jnp.float32),
        mesh=mesh,
        compiler_params=pltpu.CompilerParams(
            use_tc_tiling_on_sc=False, needs_layout_passes=False),
        scratch_types=[
            pltpu.VMEM((2 * PER_W,), jnp.int32),
            pltpu.VMEM((2 * C, DW), jnp.int32),
            pltpu.VMEM((2 * C, DW), jnp.int32),
            pltpu.VMEM((OV_LEN,), jnp.float32),
            pltpu.SemaphoreType.DMA,
            pltpu.SemaphoreType.DMA,
            pltpu.SemaphoreType.DMA,
            pltpu.SemaphoreType.DMA,
        ],
    )

    def to_packed(t):
        u = jax.lax.bitcast_convert_type(t, jnp.uint32)
        r = (u + 0x7FFF + ((u >> 16) & 1)) >> 16
        packed = r[:, :DW] | (r[:, DW:] << 16)
        return jax.lax.bitcast_convert_type(packed, jnp.int32)

    tab = jnp.concatenate([to_packed(x_user), to_packed(x_movie)], axis=0)
    idx0 = edge_label_index[0]
    idx1 = edge_label_index[1] + V
    eli = jnp.concatenate([idx0, idx1])
    return run(tab, eli)


# speedup vs baseline: 1.1728x; 1.1728x over previous
import jax
import jax.numpy as jnp
from jax import lax
from jax.experimental import pallas as pl
from jax.experimental.pallas import tpu as pltpu
from jax.experimental.pallas import tpu_sc as plsc

E = 160000
V = 10000
D = 256
DW = D // 2
NC = 2
NS = 16
NW = NC * NS
PER_W = E // NW
C = 192
NFULL = PER_W // C
TAIL = PER_W - NFULL * C
LOOP_CHUNKS = NFULL - 2
G = C // 16
OV_LEN = NFULL * C + 16


def _dot_group(buf, row0, mbase):
    lane = lax.iota(jnp.int32, 16)
    zero = jnp.zeros((16,), jnp.float32)

    def jbody(j, res):
        e = row0 + j
        me = mbase + e
        a0 = zero
        a1 = zero
        for s in range(DW // 32):
            u0 = plsc.bitcast(buf[e, pl.ds(32 * s, 16)], jnp.bfloat16)
            m0 = plsc.bitcast(buf[me, pl.ds(32 * s, 16)], jnp.bfloat16)
            u1 = plsc.bitcast(buf[e, pl.ds(32 * s + 16, 16)], jnp.bfloat16)
            m1 = plsc.bitcast(buf[me, pl.ds(32 * s + 16, 16)], jnp.bfloat16)
            q = u0 * m0 + u1 * m1
            pa, pb = plsc.unpack(q, format=plsc.PackFormat.INTERLEAVED)
            a0 = a0 + pa
            a1 = a1 + pb
        acc = a0 + a1
        for k in (1, 2, 4, 8):
            acc = acc + jnp.take_along_axis(acc, lane ^ k, axis=0,
                                            mode="promise_in_bounds")
        return jnp.where(lane == j, acc, res)

    return lax.fori_loop(0, 16, jbody, zero, unroll=2)


def _compute_chunk(buf, ov, out_off):
    def body(g, carry):
        ov[pl.ds(out_off + g * 16, 16)] = _dot_group(buf, g * 16, C)
        return carry

    lax.fori_loop(0, G, body, 0)


def _sc_body(tabu, tabm, eli, out, idxall, b0, b1, ov, su0, su1, sm0, sm1):
    c = lax.axis_index("c")
    s = lax.axis_index("s")
    wid = s * NC + c
    base = wid * PER_W

    bufs = (b0, b1)
    sus = (su0, su1)
    sms = (sm0, sm1)

    def fire_rows(k, b):
        off = k * C
        pltpu.async_copy(tabu.at[idxall.at[pl.ds(off, C)]],
                         bufs[b].at[pl.ds(0, C)], sus[b])
        pltpu.async_copy(tabm.at[idxall.at[pl.ds(PER_W + off, C)]],
                         bufs[b].at[pl.ds(C, C)], sms[b])

    def wait_rows(k, b):
        off = k * C
        pltpu.make_async_copy(tabu.at[idxall.at[pl.ds(off, C)]],
                              bufs[b].at[pl.ds(0, C)], sus[b]).wait()
        pltpu.make_async_copy(tabm.at[idxall.at[pl.ds(PER_W + off, C)]],
                              bufs[b].at[pl.ds(C, C)], sms[b]).wait()

    pltpu.sync_copy(eli.at[pl.ds(base, PER_W)], idxall.at[pl.ds(0, PER_W)])
    pltpu.sync_copy(eli.at[pl.ds(E + base, PER_W)],
                    idxall.at[pl.ds(PER_W, PER_W)])

    fire_rows(0, 0)

    def pair_body(p, carry):
        for half in range(2):
            k = 2 * p + half
            b = half
            nb = 1 - half
            fire_rows(k + 1, nb)
            wait_rows(k, b)
            _compute_chunk(bufs[b], ov, k * C)
        return carry

    lax.fori_loop(0, LOOP_CHUNKS // 2, pair_body, 0)

    k0 = LOOP_CHUNKS
    k1 = LOOP_CHUNKS + 1
    fire_rows(k1, 1)
    wait_rows(k0, 0)
    _compute_chunk(b0, ov, k0 * C)

    toff = NFULL * C
    pltpu.async_copy(tabu.at[idxall.at[pl.ds(toff, TAIL)]],
                     b0.at[pl.ds(0, TAIL)], su0)
    pltpu.async_copy(tabm.at[idxall.at[pl.ds(PER_W + toff, TAIL)]],
                     b0.at[pl.ds(TAIL, TAIL)], sm0)

    wait_rows(k1, 1)
    _compute_chunk(b1, ov, k1 * C)

    pltpu.make_async_copy(tabu.at[idxall.at[pl.ds(toff, TAIL)]],
                          b0.at[pl.ds(0, TAIL)], su0).wait()
    pltpu.make_async_copy(tabm.at[idxall.at[pl.ds(PER_W + toff, TAIL)]],
                          b0.at[pl.ds(TAIL, TAIL)], sm0).wait()
    ov[pl.ds(NFULL * C, 16)] = _dot_group(b0, 0, TAIL)

    pltpu.sync_copy(ov.at[pl.ds(0, PER_W)], out.at[pl.ds(base, PER_W)])


def kernel(x_user, x_movie, edge_label_index):
    mesh = plsc.VectorSubcoreMesh(core_axis_name="c", subcore_axis_name="s")
    run = pl.kernel(
        _sc_body,
        out_type=jax.ShapeDtypeStruct((E,), jnp.float32),
        mesh=mesh,
        compiler_params=pltpu.CompilerParams(
            use_tc_tiling_on_sc=False, needs_layout_passes=False),
        scratch_types=[
            pltpu.VMEM((2 * PER_W,), jnp.int32),
            pltpu.VMEM((2 * C, DW), jnp.int32),
            pltpu.VMEM((2 * C, DW), jnp.int32),
            pltpu.VMEM((OV_LEN,), jnp.float32),
            pltpu.SemaphoreType.DMA,
            pltpu.SemaphoreType.DMA,
            pltpu.SemaphoreType.DMA,
            pltpu.SemaphoreType.DMA,
        ],
    )

    def to_packed(t):
        u = jax.lax.bitcast_convert_type(t, jnp.uint32)
        r = (u + 0x7FFF + ((u >> 16) & 1)) >> 16
        packed = r[:, :DW] | (r[:, DW:] << 16)
        return jax.lax.bitcast_convert_type(packed, jnp.int32)

    return run(to_packed(x_user), to_packed(x_movie),
               edge_label_index.reshape(-1))
